# baseline (speedup 1.0000x reference)
import functools

import jax
import jax.numpy as jnp
from jax import lax
from jax.experimental import pallas as pl
from jax.experimental.pallas import tpu as pltpu
from jax.experimental.pallas import tpu_sc as plsc

_NY = 468
_NX = 468
_B = 2
_N = 131072
_C = 128

_NW = 16
_NV = _N // _NW
_NBIN = 480
_TRASH = 468
_ALIGN_PAD = _NY * (_NW - 1)
_TRASH_BASE = ((_N + _ALIGN_PAD + 15) // 16) * 16
_PERM_SIZE = _TRASH_BASE + _NW * _NV

_mesh = plsc.VectorSubcoreMesh(core_axis_name="c", subcore_axis_name="s")
_CP = pltpu.CompilerParams(
    needs_layout_passes=False, use_tc_tiling_on_sc=False)


@functools.partial(
    pl.kernel,
    mesh=_mesh,
    compiler_params=_CP,
    out_type=jax.ShapeDtypeStruct((_B, _C, _NY, _NX), jnp.float32),
    scratch_types=[
        pltpu.VMEM((_NV,), jnp.int32),
        pltpu.VMEM((_NV // 128, 128), jnp.int32),
        pltpu.VMEM((_NW, _NBIN), jnp.int32),
        pltpu.VMEM((_NBIN,), jnp.int32),
        pltpu.VMEM((_NBIN,), jnp.int32),
        pltpu.VMEM((_NBIN,), jnp.int32),
        pltpu.VMEM((32,), jnp.int32),
        pltpu.VMEM((32,), jnp.int32),
        pltpu.VMEM((16, _C), jnp.float32),
        pltpu.VMEM((16, _C), jnp.float32),
        pltpu.VMEM((16,), jnp.int32),
        pltpu.VMEM((16,), jnp.int32),
        pltpu.VMEM((_C, _NX), jnp.float32),
        pltpu.VMEM_SHARED((_PERM_SIZE,), jnp.int32),
        pltpu.VMEM_SHARED((_NW, _NBIN), jnp.int32),
        pltpu.SemaphoreType.DMA,
        pltpu.SemaphoreType.DMA,
        pltpu.SemaphoreType.DMA,
    ],
)
def _bev_kernel(r_hbm, pk_hbm, feat_hbm, out_hbm,
                cidv, pk2d, gv, hist, cursor, colmask, myoff, mycnt,
                fb0, fb1, pkb0, pkb1, tile, perm_spm, hist_spm,
                sg0, sg1, sem_s):
  cid = lax.axis_index("c")
  sid = lax.axis_index("s")
  iota = lax.iota(jnp.int32, 16)
  zeros16 = jnp.zeros((16,), jnp.float32)
  izeros16 = jnp.zeros((16,), jnp.int32)
  ones16 = jnp.ones((16,), jnp.int32)

  base = pl.multiple_of(sid * _NV, _NV)
  pltpu.sync_copy(r_hbm.at[pl.ds(base, _NV)], cidv)
  pltpu.sync_copy(pk_hbm.at[pl.ds(pl.multiple_of(sid * (_NV // 128), 64),
                                  _NV // 128)], pk2d)

  def _zero_bins(j, _):
    hist[pl.ds(pl.multiple_of(j * 16, 16), 16)] = izeros16
    return 0
  lax.fori_loop(0, _NBIN // 16, _zero_bins, 0)

  row0 = cid * _NY

  def _hist_body(v, _):
    cv = cidv[pl.ds(pl.multiple_of(v * 16, 16), 16)]
    rr = cv - row0
    inm = jnp.logical_and(rr >= 0, rr < _NY)
    rrc = jnp.where(inm, rr, _TRASH)
    rank, last = plsc.scan_count(rrc)
    old = plsc.load_gather(hist, [rrc])
    plsc.store_scatter(hist, [rrc], old + rank, mask=last)
    return 0
  lax.fori_loop(0, _NV // 16, _hist_body, 0)

  pltpu.sync_copy(hist, hist_spm.at[sid])
  plsc.subcore_barrier()

  pltpu.sync_copy(hist_spm, gv)
  lane_mine = iota == sid
  lane0 = iota == 0

  def _prefix_body(r, carry):
    cntv = plsc.load_gather(gv, [iota, jnp.full((16,), r, jnp.int32)])
    incl = plsc.cumsum(cntv)
    startv = (incl - cntv) + carry
    plsc.store_scatter(cursor, [jnp.full((16,), r, jnp.int32)], startv,
                       mask=lane_mine)
    total = incl[15]
    is_my_row = (r % 16) == sid
    i = r // 16
    idx_i = jnp.full((16,), i, jnp.int32)
    mym = jnp.logical_and(lane0, jnp.full((16,), is_my_row))
    plsc.store_scatter(myoff, [idx_i], jnp.full((16,), carry, jnp.int32),
                       mask=mym)
    plsc.store_scatter(mycnt, [idx_i], jnp.full((16,), total, jnp.int32),
                       mask=mym)
    return carry + ((total + 15) & (-16))
  lax.fori_loop(0, _NY, _prefix_body, jnp.int32(0))

  trash_start = jnp.full((16,), _TRASH_BASE, jnp.int32) + sid * _NV
  plsc.store_scatter(cursor, [jnp.full((16,), _TRASH, jnp.int32)],
                     trash_start, mask=lane0)

  def _pos_body(v, _):
    cv = cidv[pl.ds(pl.multiple_of(v * 16, 16), 16)]
    rr = cv - row0
    inm = jnp.logical_and(rr >= 0, rr < _NY)
    rrc = jnp.where(inm, rr, _TRASH)
    rank, last = plsc.scan_count(rrc)
    cur = plsc.load_gather(cursor, [rrc])
    plsc.store_scatter(cursor, [rrc], cur + rank, mask=last)
    pos = cur + rank - 1
    src = pk2d.at[v >> 3, pl.ds(pl.multiple_of((v & 7) * 16, 16), 16)]
    pltpu.async_copy(src, perm_spm.at[pos], sem_s).wait()
    return 0
  lax.fori_loop(0, _NV // 16, _pos_body, 0)
  plsc.subcore_barrier()

  def _full_zero():
    def _zero_tile(c, _):
      def _zt(j, _):
        tile[c, pl.ds(pl.multiple_of(j * 16, 16), 16)] = zeros16
        return 0
      lax.fori_loop(0, _NX // 16, _zt, 0)
      tile[c, pl.ds(_NX - 16, 16)] = zeros16
      return 0
    lax.fori_loop(0, _C, _zero_tile, 0)

    def _zero_cm(j, _):
      colmask[pl.ds(pl.multiple_of(j * 16, 16), 16)] = izeros16
      return 0
    lax.fori_loop(0, _NBIN // 16, _zero_cm, 0)

  _full_zero()

  fbs = (fb0, fb1)
  pkbs = (pkb0, pkb1)
  sgs = (sg0, sg1)

  def _row_body(i, _):
    r = sid + 16 * i

    @pl.when(r < _NY)
    def _process():
      offv = plsc.load_gather(myoff, [jnp.full((16,), i, jnp.int32)])
      cntv = plsc.load_gather(mycnt, [jnp.full((16,), i, jnp.int32)])
      off = offv[0]
      cnt = cntv[0]
      nb = (cnt + 15) >> 4

      def _issue(bi, B):
        boff = pl.multiple_of(off + bi * 16, 16)
        pltpu.sync_copy(perm_spm.at[pl.ds(boff, 16)], pkbs[B])
        pk = pkbs[B][...]
        valid = (bi * 16 + iota) < cnt
        vid = jnp.where(valid, pk >> 9, 0)
        pltpu.async_copy(feat_hbm.at[vid], fbs[B], sgs[B])

      def _wait(B):
        pltpu.make_async_copy(feat_hbm.at[izeros16], fbs[B], sgs[B]).wait()

      def _process_batch(bi, B):
        pk = pkbs[B][...]
        valid = (bi * 16 + iota) < cnt
        xx = jnp.where(valid, pk & 511, 0)
        fb = fbs[B]
        for j in range(16):
          xj = jnp.full((16,), xx[j], jnp.int32)
          vj = jnp.full((16,), (bi * 16 + j) < cnt)
          tm = plsc.load_gather(colmask, [xj]) > 0
          for g in range(8):
            cg = iota + g * 16
            fv = plsc.load_gather(fb, [jnp.full((16,), j, jnp.int32), cg])
            old = plsc.load_gather(tile, [cg, xj])
            new = jnp.where(tm, jnp.maximum(old, fv), fv)
            plsc.store_scatter(tile, [cg, xj], new, mask=vj)
          plsc.store_scatter(colmask, [xj], ones16, mask=vj)

      @pl.when(nb > 0)
      def _prime():
        _issue(0, 0)

      def _k_body(k, _):
        b1 = 2 * k + 1

        @pl.when(b1 < nb)
        def _():
          _issue(b1, 1)
        _wait(0)
        _process_batch(2 * k, 0)

        @pl.when(b1 < nb)
        def _():
          @pl.when(b1 + 1 < nb)
          def _():
            _issue(b1 + 1, 0)
          _wait(1)
          _process_batch(b1, 1)
        return 0
      lax.fori_loop(0, (nb + 1) >> 1, _k_body, 0)

      pltpu.sync_copy(tile, out_hbm.at[cid, :, r, :])

      @pl.when(cnt <= 256)
      def _rezero():
        def _rz_body(bi, _):
          boff = pl.multiple_of(off + bi * 16, 16)
          pltpu.sync_copy(perm_spm.at[pl.ds(boff, 16)], pkb0)
          pk = pkb0[...]
          valid = (bi * 16 + iota) < cnt
          xx = jnp.where(valid, pk & 511, 0)
          for j in range(16):
            xj = jnp.full((16,), xx[j], jnp.int32)
            for g in range(8):
              plsc.store_scatter(tile, [iota + g * 16, xj], zeros16)
            plsc.store_scatter(colmask, [xj], izeros16)
          return 0
        lax.fori_loop(0, nb, _rz_body, 0)

      @pl.when(cnt > 256)
      def _refull():
        _full_zero()
    return 0
  lax.fori_loop(0, 30, _row_body, 0)


def kernel(voxel_features, voxel_coords, batch_size):
  del batch_size
  b = voxel_coords[:, 0]
  y = voxel_coords[:, 2]
  x = voxel_coords[:, 3]
  r = b * _NY + y
  packed = (jnp.arange(_N, dtype=jnp.int32) << 9) | x
  pk2d = packed.reshape(_N // 128, 128)
  return _bev_kernel(r, pk2d, voxel_features)


# speedup vs baseline: 2.4642x; 2.4642x over previous
import functools

import jax
import jax.numpy as jnp
from jax import lax
from jax.experimental import pallas as pl
from jax.experimental.pallas import tpu as pltpu
from jax.experimental.pallas import tpu_sc as plsc

_NY = 468
_NX = 468
_B = 2
_N = 131072
_C = 128

_NW = 16
_NV = _N // _NW
_NBIN = 480
_TRASH = 468
_ALIGN_PAD = _NY * (_NW - 1)
_TRASH_BASE = ((_N + _ALIGN_PAD + 15) // 16) * 16
_PERM_SIZE = _TRASH_BASE + _NW * _NV

_mesh = plsc.VectorSubcoreMesh(core_axis_name="c", subcore_axis_name="s")
_CP = pltpu.CompilerParams(
    needs_layout_passes=False, use_tc_tiling_on_sc=False)


@functools.partial(
    pl.kernel,
    mesh=_mesh,
    compiler_params=_CP,
    out_type=jax.ShapeDtypeStruct((_NY, _NX, _B, _C), jnp.float32),
    scratch_types=[
        pltpu.VMEM((_NV,), jnp.int32),
        pltpu.VMEM((_NV // 128, 128), jnp.int32),
        pltpu.VMEM((_NW, _NBIN), jnp.int32),
        pltpu.VMEM((_NBIN,), jnp.int32),
        pltpu.VMEM((_NBIN,), jnp.int32),
        pltpu.VMEM((_NBIN,), jnp.int32),
        pltpu.VMEM((32,), jnp.int32),
        pltpu.VMEM((32,), jnp.int32),
        pltpu.VMEM((16, _C), jnp.float32),
        pltpu.VMEM((16, _C), jnp.float32),
        pltpu.VMEM((16,), jnp.int32),
        pltpu.VMEM((16,), jnp.int32),
        pltpu.VMEM((_NX, _C), jnp.float32),
        pltpu.VMEM_SHARED((_PERM_SIZE,), jnp.int32),
        pltpu.VMEM_SHARED((_NW, _NBIN), jnp.int32),
        pltpu.SemaphoreType.DMA,
        pltpu.SemaphoreType.DMA,
        pltpu.SemaphoreType.DMA,
    ],
)
def _bev_kernel(r_hbm, pk_hbm, feat_hbm, out_hbm,
                cidv, pk2d, gv, hist, cursor, colmask, myoff, mycnt,
                fb0, fb1, pkb0, pkb1, tile, perm_spm, hist_spm,
                sg0, sg1, sem_s):
  cid = lax.axis_index("c")
  sid = lax.axis_index("s")
  iota = lax.iota(jnp.int32, 16)
  zeros16 = jnp.zeros((16,), jnp.float32)
  izeros16 = jnp.zeros((16,), jnp.int32)
  ones16 = jnp.ones((16,), jnp.int32)

  base = pl.multiple_of(sid * _NV, _NV)
  pltpu.sync_copy(r_hbm.at[pl.ds(base, _NV)], cidv)
  pltpu.sync_copy(pk_hbm.at[pl.ds(pl.multiple_of(sid * (_NV // 128), 64),
                                  _NV // 128)], pk2d)

  def _zero_bins(j, _):
    hist[pl.ds(pl.multiple_of(j * 16, 16), 16)] = izeros16
    return 0
  lax.fori_loop(0, _NBIN // 16, _zero_bins, 0)

  row0 = cid * _NY

  def _hist_body(v, _):
    cv = cidv[pl.ds(pl.multiple_of(v * 16, 16), 16)]
    rr = cv - row0
    inm = jnp.logical_and(rr >= 0, rr < _NY)
    rrc = jnp.where(inm, rr, _TRASH)
    rank, last = plsc.scan_count(rrc)
    old = plsc.load_gather(hist, [rrc])
    plsc.store_scatter(hist, [rrc], old + rank, mask=last)
    return 0
  lax.fori_loop(0, _NV // 16, _hist_body, 0)

  pltpu.sync_copy(hist, hist_spm.at[sid])
  plsc.subcore_barrier()

  pltpu.sync_copy(hist_spm, gv)
  lane_mine = iota == sid
  lane0 = iota == 0

  def _prefix_body(r, carry):
    cntv = plsc.load_gather(gv, [iota, jnp.full((16,), r, jnp.int32)])
    incl = plsc.cumsum(cntv)
    startv = (incl - cntv) + carry
    plsc.store_scatter(cursor, [jnp.full((16,), r, jnp.int32)], startv,
                       mask=lane_mine)
    total = incl[15]
    is_my_row = (r % 16) == sid
    i = r // 16
    idx_i = jnp.full((16,), i, jnp.int32)
    mym = jnp.logical_and(lane0, jnp.full((16,), is_my_row))
    plsc.store_scatter(myoff, [idx_i], jnp.full((16,), carry, jnp.int32),
                       mask=mym)
    plsc.store_scatter(mycnt, [idx_i], jnp.full((16,), total, jnp.int32),
                       mask=mym)
    return carry + ((total + 15) & (-16))
  lax.fori_loop(0, _NY, _prefix_body, jnp.int32(0))

  trash_start = jnp.full((16,), _TRASH_BASE, jnp.int32) + sid * _NV
  plsc.store_scatter(cursor, [jnp.full((16,), _TRASH, jnp.int32)],
                     trash_start, mask=lane0)

  def _pos_body(v, _):
    cv = cidv[pl.ds(pl.multiple_of(v * 16, 16), 16)]
    rr = cv - row0
    inm = jnp.logical_and(rr >= 0, rr < _NY)
    rrc = jnp.where(inm, rr, _TRASH)
    rank, last = plsc.scan_count(rrc)
    cur = plsc.load_gather(cursor, [rrc])
    plsc.store_scatter(cursor, [rrc], cur + rank, mask=last)
    pos = cur + rank - 1
    src = pk2d.at[v >> 3, pl.ds(pl.multiple_of((v & 7) * 16, 16), 16)]
    pltpu.async_copy(src, perm_spm.at[pos], sem_s).wait()
    return 0
  lax.fori_loop(0, _NV // 16, _pos_body, 0)
  plsc.subcore_barrier()

  def _full_zero():
    def _zero_tile(x, _):
      def _zt(j, _):
        tile[x, pl.ds(pl.multiple_of(j * 16, 16), 16)] = zeros16
        return 0
      lax.fori_loop(0, _C // 16, _zt, 0)
      return 0
    lax.fori_loop(0, _NX, _zero_tile, 0)

    def _zero_cm(j, _):
      colmask[pl.ds(pl.multiple_of(j * 16, 16), 16)] = izeros16
      return 0
    lax.fori_loop(0, _NBIN // 16, _zero_cm, 0)

  _full_zero()

  fbs = (fb0, fb1)
  pkbs = (pkb0, pkb1)
  sgs = (sg0, sg1)

  def _row_body(i, _):
    r = sid + 16 * i

    @pl.when(r < _NY)
    def _process():
      offv = plsc.load_gather(myoff, [jnp.full((16,), i, jnp.int32)])
      cntv = plsc.load_gather(mycnt, [jnp.full((16,), i, jnp.int32)])
      off = offv[0]
      cnt = cntv[0]
      nb = (cnt + 15) >> 4

      def _issue(bi, B):
        boff = pl.multiple_of(off + bi * 16, 16)
        pltpu.sync_copy(perm_spm.at[pl.ds(boff, 16)], pkbs[B])
        pk = pkbs[B][...]
        valid = (bi * 16 + iota) < cnt
        vid = jnp.where(valid, pk >> 9, 0)
        pltpu.async_copy(feat_hbm.at[vid], fbs[B], sgs[B])

      def _wait(B):
        pltpu.make_async_copy(feat_hbm.at[izeros16], fbs[B], sgs[B]).wait()

      def _process_batch(bi, B):
        pk = pkbs[B][...]
        valid = (bi * 16 + iota) < cnt
        xx = jnp.where(valid, pk & 511, 0)
        fb = fbs[B]
        for j in range(16):
          xj = jnp.full((16,), xx[j], jnp.int32)
          vj = jnp.full((16,), (bi * 16 + j) < cnt)
          tm = plsc.load_gather(colmask, [xj]) > 0
          for g in range(8):
            cg = iota + g * 16
            fv = plsc.load_gather(fb, [jnp.full((16,), j, jnp.int32), cg])
            old = plsc.load_gather(tile, [xj, cg])
            new = jnp.where(tm, jnp.maximum(old, fv), fv)
            plsc.store_scatter(tile, [xj, cg], new, mask=vj)
          plsc.store_scatter(colmask, [xj], ones16, mask=vj)

      @pl.when(nb > 0)
      def _prime():
        _issue(0, 0)

      def _k_body(k, _):
        b1 = 2 * k + 1

        @pl.when(b1 < nb)
        def _():
          _issue(b1, 1)
        _wait(0)
        _process_batch(2 * k, 0)

        @pl.when(b1 < nb)
        def _():
          @pl.when(b1 + 1 < nb)
          def _():
            _issue(b1 + 1, 0)
          _wait(1)
          _process_batch(b1, 1)
        return 0
      lax.fori_loop(0, (nb + 1) >> 1, _k_body, 0)

      pltpu.sync_copy(tile, out_hbm.at[r, :, cid, :])

      @pl.when(cnt <= 256)
      def _rezero():
        def _rz_body(bi, _):
          boff = pl.multiple_of(off + bi * 16, 16)
          pltpu.sync_copy(perm_spm.at[pl.ds(boff, 16)], pkb0)
          pk = pkb0[...]
          valid = (bi * 16 + iota) < cnt
          xx = jnp.where(valid, pk & 511, 0)
          for j in range(16):
            xj = jnp.full((16,), xx[j], jnp.int32)
            for g in range(8):
              plsc.store_scatter(tile, [xj, iota + g * 16], zeros16)
            plsc.store_scatter(colmask, [xj], izeros16)
          return 0
        lax.fori_loop(0, nb, _rz_body, 0)

      @pl.when(cnt > 256)
      def _refull():
        _full_zero()
    return 0
  lax.fori_loop(0, 30, _row_body, 0)


def kernel(voxel_features, voxel_coords, batch_size):
  del batch_size
  b = voxel_coords[:, 0]
  y = voxel_coords[:, 2]
  x = voxel_coords[:, 3]
  r = b * _NY + y
  packed = (jnp.arange(_N, dtype=jnp.int32) << 9) | x
  pk2d = packed.reshape(_N // 128, 128)
  out_yxbc = _bev_kernel(r, pk2d, voxel_features)
  return out_yxbc.transpose(2, 3, 0, 1)


# speedup vs baseline: 2.6403x; 1.0715x over previous
import functools

import jax
import jax.numpy as jnp
from jax import lax
from jax.experimental import pallas as pl
from jax.experimental.pallas import tpu as pltpu
from jax.experimental.pallas import tpu_sc as plsc

_NY = 468
_NX = 468
_B = 2
_N = 131072
_C = 128

_NW = 16
_NV = _N // _NW
_NBIN = 480
_TRASH = 468
_ALIGN_PAD = _NY * (_NW - 1)
_TRASH_BASE = ((_N + _ALIGN_PAD + 15) // 16) * 16
_PERM_SIZE = _TRASH_BASE + _NW * _NV

_mesh = plsc.VectorSubcoreMesh(core_axis_name="c", subcore_axis_name="s")
_CP = pltpu.CompilerParams(
    needs_layout_passes=False, use_tc_tiling_on_sc=False)


@functools.partial(
    pl.kernel,
    mesh=_mesh,
    compiler_params=_CP,
    out_type=jax.ShapeDtypeStruct((_NY, _NX, _B, _C), jnp.float32),
    scratch_types=[
        pltpu.VMEM((_NV,), jnp.int32),
        pltpu.VMEM((_NV // 128, 128), jnp.int32),
        pltpu.VMEM((_NW, _NBIN), jnp.int32),
        pltpu.VMEM((_NBIN,), jnp.int32),
        pltpu.VMEM((_NBIN,), jnp.int32),
        pltpu.VMEM((_NBIN,), jnp.int32),
        pltpu.VMEM((32,), jnp.int32),
        pltpu.VMEM((32,), jnp.int32),
        pltpu.VMEM((16, _C), jnp.float32),
        pltpu.VMEM((16, _C), jnp.float32),
        pltpu.VMEM((16,), jnp.int32),
        pltpu.VMEM((16,), jnp.int32),
        pltpu.VMEM((_NX, _C), jnp.float32),
        pltpu.VMEM_SHARED((_PERM_SIZE,), jnp.int32),
        pltpu.VMEM_SHARED((_NW, _NBIN), jnp.int32),
        pltpu.SemaphoreType.DMA,
        pltpu.SemaphoreType.DMA,
        pltpu.SemaphoreType.DMA,
    ],
)
def _bev_kernel(r_hbm, pk_hbm, feat_hbm, out_hbm,
                cidv, pk2d, gv, hist, cursor, colmask, myoff, mycnt,
                fb0, fb1, pkb0, pkb1, tile, perm_spm, hist_spm,
                sg0, sg1, sem_s):
  cid = lax.axis_index("c")
  sid = lax.axis_index("s")
  iota = lax.iota(jnp.int32, 16)
  zeros16 = jnp.zeros((16,), jnp.float32)
  izeros16 = jnp.zeros((16,), jnp.int32)
  ones16 = jnp.ones((16,), jnp.int32)

  base = pl.multiple_of(sid * _NV, _NV)
  pltpu.sync_copy(r_hbm.at[pl.ds(base, _NV)], cidv)
  pltpu.sync_copy(pk_hbm.at[pl.ds(pl.multiple_of(sid * (_NV // 128), 64),
                                  _NV // 128)], pk2d)

  def _zero_bins(j, _):
    hist[pl.ds(pl.multiple_of(j * 16, 16), 16)] = izeros16
    return 0
  lax.fori_loop(0, _NBIN // 16, _zero_bins, 0)

  row0 = cid * _NY

  def _hist_body(v, _):
    cv = cidv[pl.ds(pl.multiple_of(v * 16, 16), 16)]
    rr = cv - row0
    inm = jnp.logical_and(rr >= 0, rr < _NY)
    rrc = jnp.where(inm, rr, _TRASH)
    rank, last = plsc.scan_count(rrc)
    old = plsc.load_gather(hist, [rrc])
    plsc.store_scatter(hist, [rrc], old + rank, mask=last)
    return 0
  lax.fori_loop(0, _NV // 16, _hist_body, 0)

  pltpu.sync_copy(hist, hist_spm.at[sid])
  plsc.subcore_barrier()

  pltpu.sync_copy(hist_spm, gv)
  lane_mine = iota == sid
  lane0 = iota == 0

  def _prefix_body(r, carry):
    cntv = plsc.load_gather(gv, [iota, jnp.full((16,), r, jnp.int32)])
    incl = plsc.cumsum(cntv)
    startv = (incl - cntv) + carry
    plsc.store_scatter(cursor, [jnp.full((16,), r, jnp.int32)], startv,
                       mask=lane_mine)
    total = incl[15]
    is_my_row = (r % 16) == sid
    i = r // 16
    idx_i = jnp.full((16,), i, jnp.int32)
    mym = jnp.logical_and(lane0, jnp.full((16,), is_my_row))
    plsc.store_scatter(myoff, [idx_i], jnp.full((16,), carry, jnp.int32),
                       mask=mym)
    plsc.store_scatter(mycnt, [idx_i], jnp.full((16,), total, jnp.int32),
                       mask=mym)
    return carry + ((total + 15) & (-16))
  lax.fori_loop(0, _NY, _prefix_body, jnp.int32(0))

  trash_start = jnp.full((16,), _TRASH_BASE, jnp.int32) + sid * _NV
  plsc.store_scatter(cursor, [jnp.full((16,), _TRASH, jnp.int32)],
                     trash_start, mask=lane0)

  def _pos_body(v, _):
    cv = cidv[pl.ds(pl.multiple_of(v * 16, 16), 16)]
    rr = cv - row0
    inm = jnp.logical_and(rr >= 0, rr < _NY)
    rrc = jnp.where(inm, rr, _TRASH)
    rank, last = plsc.scan_count(rrc)
    cur = plsc.load_gather(cursor, [rrc])
    plsc.store_scatter(cursor, [rrc], cur + rank, mask=last)
    pos = cur + rank - 1
    src = pk2d.at[v >> 3, pl.ds(pl.multiple_of((v & 7) * 16, 16), 16)]
    pltpu.async_copy(src, perm_spm.at[pos], sem_s).wait()
    return 0
  lax.fori_loop(0, _NV // 16, _pos_body, 0)
  plsc.subcore_barrier()

  def _full_zero():
    def _zero_tile(x, _):
      def _zt(j, _):
        tile[x, pl.ds(pl.multiple_of(j * 16, 16), 16)] = zeros16
        return 0
      lax.fori_loop(0, _C // 16, _zt, 0)
      return 0
    lax.fori_loop(0, _NX, _zero_tile, 0)

    def _zero_cm(j, _):
      colmask[pl.ds(pl.multiple_of(j * 16, 16), 16)] = izeros16
      return 0
    lax.fori_loop(0, _NBIN // 16, _zero_cm, 0)

  _full_zero()

  fbs = (fb0, fb1)
  pkbs = (pkb0, pkb1)
  sgs = (sg0, sg1)

  def _row_body(i, _):
    r = sid + 16 * i

    @pl.when(r < _NY)
    def _process():
      offv = plsc.load_gather(myoff, [jnp.full((16,), i, jnp.int32)])
      cntv = plsc.load_gather(mycnt, [jnp.full((16,), i, jnp.int32)])
      off = offv[0]
      cnt = cntv[0]
      nb = (cnt + 15) >> 4

      def _issue(bi, B):
        boff = pl.multiple_of(off + bi * 16, 16)
        pltpu.sync_copy(perm_spm.at[pl.ds(boff, 16)], pkbs[B])
        pk = pkbs[B][...]
        valid = (bi * 16 + iota) < cnt
        vid = jnp.where(valid, pk >> 9, 0)
        pltpu.async_copy(feat_hbm.at[vid], fbs[B], sgs[B])

      def _wait(B):
        pltpu.make_async_copy(feat_hbm.at[izeros16], fbs[B], sgs[B]).wait()

      def _process_batch(bi, B):
        pk = pkbs[B][...]
        valid = (bi * 16 + iota) < cnt
        xx = jnp.where(valid, pk & 511, 0)
        fb = fbs[B]
        for j in range(16):
          xs = xx[j]

          @pl.when((bi * 16 + j) < cnt)
          def _upd(xs=xs, j=j, fb=fb):
            xj = jnp.full((16,), xs, jnp.int32)
            tm = plsc.load_gather(colmask, [xj]) > 0
            for g in range(8):
              fv = fb[j, pl.ds(g * 16, 16)]
              old = tile[xs, pl.ds(g * 16, 16)]
              tile[xs, pl.ds(g * 16, 16)] = jnp.where(
                  tm, jnp.maximum(old, fv), fv)
            plsc.store_scatter(colmask, [xj], ones16)

      @pl.when(nb > 0)
      def _prime():
        _issue(0, 0)

      def _k_body(k, _):
        b1 = 2 * k + 1

        @pl.when(b1 < nb)
        def _():
          _issue(b1, 1)
        _wait(0)
        _process_batch(2 * k, 0)

        @pl.when(b1 < nb)
        def _():
          @pl.when(b1 + 1 < nb)
          def _():
            _issue(b1 + 1, 0)
          _wait(1)
          _process_batch(b1, 1)
        return 0
      lax.fori_loop(0, (nb + 1) >> 1, _k_body, 0)

      pltpu.sync_copy(tile, out_hbm.at[r, :, cid, :])

      @pl.when(cnt <= 256)
      def _rezero():
        def _rz_body(bi, _):
          boff = pl.multiple_of(off + bi * 16, 16)
          pltpu.sync_copy(perm_spm.at[pl.ds(boff, 16)], pkb0)
          pk = pkb0[...]
          valid = (bi * 16 + iota) < cnt
          xx = jnp.where(valid, pk & 511, 0)
          for j in range(16):
            xs = xx[j]
            for g in range(8):
              tile[xs, pl.ds(g * 16, 16)] = zeros16
            plsc.store_scatter(colmask, [jnp.full((16,), xs, jnp.int32)],
                               izeros16)
          return 0
        lax.fori_loop(0, nb, _rz_body, 0)

      @pl.when(cnt > 256)
      def _refull():
        _full_zero()
    return 0
  lax.fori_loop(0, 30, _row_body, 0)


def kernel(voxel_features, voxel_coords, batch_size):
  del batch_size
  b = voxel_coords[:, 0]
  y = voxel_coords[:, 2]
  x = voxel_coords[:, 3]
  r = b * _NY + y
  packed = (jnp.arange(_N, dtype=jnp.int32) << 9) | x
  pk2d = packed.reshape(_N // 128, 128)
  out_yxbc = _bev_kernel(r, pk2d, voxel_features)
  return out_yxbc.transpose(2, 3, 0, 1)
